# Initial kernel scaffold; baseline (speedup 1.0000x reference)
import jax
import jax.numpy as jnp
from jax.experimental import pallas as pl


def kernel(x, W_in, b_in, W_out, b_out, codebook):
    raise NotImplementedError("write your pallas kernel here")


# speedup vs baseline: 1.2850x; 1.2850x over previous
import jax
import jax.numpy as jnp
from jax import lax
from jax.experimental import pallas as pl
from jax.experimental.pallas import tpu as pltpu
from jax.experimental.pallas import tpu_sc as plsc

B, C, N, D = 8, 16, 256, 256
E = 768
K = 4096
T = B * C * N
TM = 256
KT = 512

NC, NS = 2, 16
NW = NC * NS
TW = T // NW
CH = 64


def _prep_body(cb_ref, wout_ref, bout_ref, cbb_ref, cn_ref, cbw_ref):
    c = cb_ref[...]
    cbb_ref[...] = c.astype(jnp.bfloat16)
    cn_ref[...] = jnp.sum(c * c, axis=1, keepdims=True)
    cbw = lax.dot_general(c.astype(jnp.bfloat16),
                          wout_ref[...].astype(jnp.bfloat16),
                          (((1,), (1,)), ((), ())),
                          preferred_element_type=jnp.float32)
    cbw_ref[...] = cbw + bout_ref[...]


def _vq_body(x_ref, win_ref, bin_ref, cbb_ref, cn_ref,
             idx_ref, aux_ref, acc_ref):
    i = pl.program_id(0)
    xb = x_ref[...].astype(jnp.bfloat16)
    wb = win_ref[...].astype(jnp.bfloat16)
    proj = lax.dot_general(xb, wb, (((1,), (1,)), ((), ())),
                           preferred_element_type=jnp.float32)
    proj = proj + bin_ref[...]
    a = jnp.sum(proj * proj, axis=1, keepdims=True)
    fb = proj.astype(jnp.bfloat16)
    mm = lax.dot_general(fb, cbb_ref[...], (((1,), (1,)), ((), ())),
                         preferred_element_type=jnp.float32)
    d2 = (a - 2.0 * mm) + cn_ref[...]
    m = jnp.min(d2, axis=1, keepdims=True)
    cols = lax.broadcasted_iota(jnp.int32, (TM, K), 1)
    idx_ref[...] = jnp.min(jnp.where(d2 == m, cols, K), axis=1, keepdims=True)

    @pl.when(i == 0)
    def _():
        acc_ref[0] = 0.0

    acc_ref[0] += jnp.sum(m)

    @pl.when(i == pl.num_programs(0) - 1)
    def _():
        aux_ref[...] = jnp.full((1, 1), acc_ref[0] * (1.0 / (T * E)),
                                dtype=jnp.float32)


def _gather2_body(idx_hbm, cb_hbm, cbw_hbm, q_hbm, o_hbm,
                  idx_v, qrows, orows, sem1, sem2):
    wid = lax.axis_index("s") * NC + lax.axis_index("c")
    base = wid * TW
    pltpu.sync_copy(idx_hbm.at[pl.ds(base, TW)], idx_v)
    for c in range(TW // CH):
        ic = idx_v.at[pl.ds(c * CH, CH)]
        pltpu.async_copy(cb_hbm.at[ic], qrows, sem1).wait()
        pltpu.async_copy(cbw_hbm.at[ic], orows, sem2).wait()
        pltpu.sync_copy(qrows, q_hbm.at[pl.ds(base + c * CH, CH)])
        pltpu.sync_copy(orows, o_hbm.at[pl.ds(base + c * CH, CH)])


def kernel(x, W_in, b_in, W_out, b_out, codebook):
    x2d = x.reshape(T, D)
    bin2d = b_in.reshape(1, E)
    bout2d = b_out.reshape(1, D)

    cbb, cn_col, cbw = pl.pallas_call(
        _prep_body,
        grid=(K // KT,),
        in_specs=[
            pl.BlockSpec((KT, E), lambda i: (i, 0)),
            pl.BlockSpec((D, E), lambda i: (0, 0)),
            pl.BlockSpec((1, D), lambda i: (0, 0)),
        ],
        out_specs=[
            pl.BlockSpec((KT, E), lambda i: (i, 0)),
            pl.BlockSpec((KT, 1), lambda i: (i, 0)),
            pl.BlockSpec((KT, D), lambda i: (i, 0)),
        ],
        out_shape=[
            jax.ShapeDtypeStruct((K, E), jnp.bfloat16),
            jax.ShapeDtypeStruct((K, 1), jnp.float32),
            jax.ShapeDtypeStruct((K, D), jnp.float32),
        ],
    )(codebook, W_out, bout2d)

    cn_row = cn_col.reshape(1, K)

    idx2d, aux = pl.pallas_call(
        _vq_body,
        grid=(T // TM,),
        in_specs=[
            pl.BlockSpec((TM, D), lambda i: (i, 0)),
            pl.BlockSpec((E, D), lambda i: (0, 0)),
            pl.BlockSpec((1, E), lambda i: (0, 0)),
            pl.BlockSpec((K, E), lambda i: (0, 0)),
            pl.BlockSpec((1, K), lambda i: (0, 0)),
        ],
        out_specs=[
            pl.BlockSpec((TM, 1), lambda i: (i, 0)),
            pl.BlockSpec((1, 1), lambda i: (0, 0)),
        ],
        out_shape=[
            jax.ShapeDtypeStruct((T, 1), jnp.int32),
            jax.ShapeDtypeStruct((1, 1), jnp.float32),
        ],
        scratch_shapes=[pltpu.SMEM((1,), jnp.float32)],
        compiler_params=pltpu.CompilerParams(
            dimension_semantics=("arbitrary",),
        ),
    )(x2d, W_in, bin2d, cbb, cn_row)

    idx1d = idx2d.reshape(T)

    gather2 = pl.kernel(
        _gather2_body,
        out_type=[
            jax.ShapeDtypeStruct((T, E), jnp.float32),
            jax.ShapeDtypeStruct((T, D), jnp.float32),
        ],
        mesh=plsc.VectorSubcoreMesh(
            core_axis_name="c", subcore_axis_name="s",
            num_cores=NC, num_subcores=NS,
        ),
        scratch_types=[
            pltpu.VMEM((TW,), jnp.int32),
            pltpu.VMEM((CH, E), jnp.float32),
            pltpu.VMEM((CH, D), jnp.float32),
            pltpu.SemaphoreType.DMA,
            pltpu.SemaphoreType.DMA,
        ],
    )
    quant2d, out2d = gather2(idx1d, codebook, cbw)

    out = out2d.reshape(B, C, N, D)
    indices = idx1d.reshape(B, C * N)
    quantized = quant2d.reshape(B, C, N, E)
    aux_loss = aux.reshape(())
    return out, indices, quantized, aux_loss


# speedup vs baseline: 1.4269x; 1.1104x over previous
import jax
import jax.numpy as jnp
from jax import lax
from jax.experimental import pallas as pl
from jax.experimental.pallas import tpu as pltpu
from jax.experimental.pallas import tpu_sc as plsc

B, C, N, D = 8, 16, 256, 256
E = 768
K = 4096
T = B * C * N
TM = 256
KT = 512

NC, NS = 2, 16
NW = NC * NS
TW = T // NW
CH = 64


def _prep_body(cb_ref, wout_ref, bout_ref, cbb_ref, cn_ref, cbw_ref):
    c = cb_ref[...]
    cbb_ref[...] = c.astype(jnp.bfloat16)
    cn_ref[...] = jnp.sum(c * c, axis=1, keepdims=True)
    cbw = lax.dot_general(c.astype(jnp.bfloat16),
                          wout_ref[...].astype(jnp.bfloat16),
                          (((1,), (1,)), ((), ())),
                          preferred_element_type=jnp.float32)
    cbw_ref[...] = cbw + bout_ref[...]


def _vq_body(x_ref, win_ref, bin_ref, cbb_ref, cn_ref,
             idx_ref, aux_ref, acc_ref):
    i = pl.program_id(0)
    xb = x_ref[...].astype(jnp.bfloat16)
    wb = win_ref[...].astype(jnp.bfloat16)
    proj = lax.dot_general(xb, wb, (((1,), (1,)), ((), ())),
                           preferred_element_type=jnp.float32)
    proj = proj + bin_ref[...]
    a = jnp.sum(proj * proj, axis=1, keepdims=True)
    fb2 = (proj + proj).astype(jnp.bfloat16)
    KC = K // 4
    bv = None
    bi = None
    for j in range(4):
        mm2 = lax.dot_general(fb2, cbb_ref[pl.ds(j * KC, KC), :],
                              (((1,), (1,)), ((), ())),
                              preferred_element_type=jnp.float32)
        d2 = (a - mm2) + cn_ref[:, pl.ds(j * KC, KC)]
        m = jnp.min(d2, axis=1, keepdims=True)
        cols = lax.broadcasted_iota(jnp.int32, (TM, KC), 1) + (j * KC)
        ii = jnp.min(jnp.where(d2 == m, cols, K), axis=1, keepdims=True)
        if j == 0:
            bv, bi = m, ii
        else:
            take = m < bv
            bi = jnp.where(take, ii, bi)
            bv = jnp.minimum(bv, m)
    idx_ref[...] = bi

    @pl.when(i == 0)
    def _():
        acc_ref[0] = 0.0

    acc_ref[0] += jnp.sum(bv)

    @pl.when(i == pl.num_programs(0) - 1)
    def _():
        aux_ref[...] = jnp.full((1, 1), acc_ref[0] * (1.0 / (T * E)),
                                dtype=jnp.float32)


def _gather2_body(idx_hbm, cb_hbm, cbw_hbm, q_hbm, o_hbm,
                  idx_v, qrows, orows, sem1, sem2):
    wid = lax.axis_index("s") * NC + lax.axis_index("c")
    base = wid * TW
    pltpu.sync_copy(idx_hbm.at[pl.ds(base, TW)], idx_v)
    for c in range(TW // CH):
        ic = idx_v.at[pl.ds(c * CH, CH)]
        d1 = pltpu.async_copy(cb_hbm.at[ic], qrows, sem1)
        d2 = pltpu.async_copy(cbw_hbm.at[ic], orows, sem2)
        d1.wait()
        d2.wait()
        pltpu.sync_copy(qrows, q_hbm.at[pl.ds(base + c * CH, CH)])
        pltpu.sync_copy(orows, o_hbm.at[pl.ds(base + c * CH, CH)])


def kernel(x, W_in, b_in, W_out, b_out, codebook):
    x2d = x.reshape(T, D)
    bin2d = b_in.reshape(1, E)
    bout2d = b_out.reshape(1, D)

    cbb, cn_col, cbw = pl.pallas_call(
        _prep_body,
        grid=(K // KT,),
        in_specs=[
            pl.BlockSpec((KT, E), lambda i: (i, 0)),
            pl.BlockSpec((D, E), lambda i: (0, 0)),
            pl.BlockSpec((1, D), lambda i: (0, 0)),
        ],
        out_specs=[
            pl.BlockSpec((KT, E), lambda i: (i, 0)),
            pl.BlockSpec((KT, 1), lambda i: (i, 0)),
            pl.BlockSpec((KT, D), lambda i: (i, 0)),
        ],
        out_shape=[
            jax.ShapeDtypeStruct((K, E), jnp.bfloat16),
            jax.ShapeDtypeStruct((K, 1), jnp.float32),
            jax.ShapeDtypeStruct((K, D), jnp.float32),
        ],
    )(codebook, W_out, bout2d)

    cn_row = cn_col.reshape(1, K)

    idx2d, aux = pl.pallas_call(
        _vq_body,
        grid=(T // TM,),
        in_specs=[
            pl.BlockSpec((TM, D), lambda i: (i, 0)),
            pl.BlockSpec((E, D), lambda i: (0, 0)),
            pl.BlockSpec((1, E), lambda i: (0, 0)),
            pl.BlockSpec((K, E), lambda i: (0, 0)),
            pl.BlockSpec((1, K), lambda i: (0, 0)),
        ],
        out_specs=[
            pl.BlockSpec((TM, 1), lambda i: (i, 0)),
            pl.BlockSpec((1, 1), lambda i: (0, 0)),
        ],
        out_shape=[
            jax.ShapeDtypeStruct((T, 1), jnp.int32),
            jax.ShapeDtypeStruct((1, 1), jnp.float32),
        ],
        scratch_shapes=[pltpu.SMEM((1,), jnp.float32)],
        compiler_params=pltpu.CompilerParams(
            dimension_semantics=("arbitrary",),
        ),
    )(x2d, W_in, bin2d, cbb, cn_row)

    idx1d = idx2d.reshape(T)

    gather2 = pl.kernel(
        _gather2_body,
        out_type=[
            jax.ShapeDtypeStruct((T, E), jnp.float32),
            jax.ShapeDtypeStruct((T, D), jnp.float32),
        ],
        mesh=plsc.VectorSubcoreMesh(
            core_axis_name="c", subcore_axis_name="s",
            num_cores=NC, num_subcores=NS,
        ),
        scratch_types=[
            pltpu.VMEM((TW,), jnp.int32),
            pltpu.VMEM((CH, E), jnp.float32),
            pltpu.VMEM((CH, D), jnp.float32),
            pltpu.SemaphoreType.DMA,
            pltpu.SemaphoreType.DMA,
        ],
    )
    quant2d, out2d = gather2(idx1d, codebook, cbw)

    out = out2d.reshape(B, C, N, D)
    indices = idx1d.reshape(B, C * N)
    quantized = quant2d.reshape(B, C, N, E)
    aux_loss = aux.reshape(())
    return out, indices, quantized, aux_loss


# speedup vs baseline: 1.4276x; 1.0005x over previous
import jax
import jax.numpy as jnp
from jax import lax
from jax.experimental import pallas as pl
from jax.experimental.pallas import tpu as pltpu
from jax.experimental.pallas import tpu_sc as plsc

B, C, N, D = 8, 16, 256, 256
E = 768
K = 4096
T = B * C * N
TM = 256
KT = 512

NC, NS = 2, 16
NW = NC * NS
TW = T // NW
CH = 16
NBUF = 4


def _prep_body(cb_ref, wout_ref, bout_ref, cbb_ref, cn_ref, cbw_ref):
    c = cb_ref[...]
    cbb_ref[...] = c.astype(jnp.bfloat16)
    cn_ref[...] = jnp.sum(c * c, axis=1, keepdims=True)
    cbw = lax.dot_general(c.astype(jnp.bfloat16),
                          wout_ref[...].astype(jnp.bfloat16),
                          (((1,), (1,)), ((), ())),
                          preferred_element_type=jnp.float32)
    cbw_ref[...] = cbw + bout_ref[...]


def _vq_body(x_ref, win_ref, bin_ref, cbb_ref, cn_ref,
             idx_ref, aux_ref, acc_ref):
    i = pl.program_id(0)
    xb = x_ref[...].astype(jnp.bfloat16)
    wb = win_ref[...].astype(jnp.bfloat16)
    proj = lax.dot_general(xb, wb, (((1,), (1,)), ((), ())),
                           preferred_element_type=jnp.float32)
    proj = proj + bin_ref[...]
    a = jnp.sum(proj * proj, axis=1, keepdims=True)
    fb2 = (proj + proj).astype(jnp.bfloat16)
    KC = K // 4
    bv = None
    bi = None
    for j in range(4):
        mm2 = lax.dot_general(fb2, cbb_ref[pl.ds(j * KC, KC), :],
                              (((1,), (1,)), ((), ())),
                              preferred_element_type=jnp.float32)
        d2 = (a - mm2) + cn_ref[:, pl.ds(j * KC, KC)]
        m = jnp.min(d2, axis=1, keepdims=True)
        cols = lax.broadcasted_iota(jnp.int32, (TM, KC), 1) + (j * KC)
        ii = jnp.min(jnp.where(d2 == m, cols, K), axis=1, keepdims=True)
        if j == 0:
            bv, bi = m, ii
        else:
            take = m < bv
            bi = jnp.where(take, ii, bi)
            bv = jnp.minimum(bv, m)
    idx_ref[...] = bi

    @pl.when(i == 0)
    def _():
        acc_ref[0] = 0.0

    acc_ref[0] += jnp.sum(bv)

    @pl.when(i == pl.num_programs(0) - 1)
    def _():
        aux_ref[...] = jnp.full((1, 1), acc_ref[0] * (1.0 / (T * E)),
                                dtype=jnp.float32)


def _gather2_body(idx_hbm, cb_hbm, cbw_hbm, q_hbm, o_hbm,
                  idx_v, q0, q1, q2, q3, o0, o1, o2, o3,
                  sq0, sq1, sq2, sq3, so0, so1, so2, so3):
    wid = lax.axis_index("s") * NC + lax.axis_index("c")
    base = wid * TW
    pltpu.sync_copy(idx_hbm.at[pl.ds(base, TW)], idx_v)
    qb, ob = (q0, q1, q2, q3), (o0, o1, o2, o3)
    sq, so = (sq0, sq1, sq2, sq3), (so0, so1, so2, so3)

    def body(g, carry):
        c = g * (NBUF * CH)
        descs = []
        for k in range(NBUF):
            ic = idx_v.at[pl.ds(c + k * CH, CH)]
            descs.append((pltpu.async_copy(cb_hbm.at[ic], qb[k], sq[k]),
                          pltpu.async_copy(cbw_hbm.at[ic], ob[k], so[k])))
        for k in range(NBUF):
            d1, d2 = descs[k]
            d1.wait()
            d2.wait()
            off = base + c + k * CH
            pltpu.sync_copy(qb[k], q_hbm.at[pl.ds(off, CH)])
            pltpu.sync_copy(ob[k], o_hbm.at[pl.ds(off, CH)])
        return carry

    lax.fori_loop(0, TW // (NBUF * CH), body, 0)


def kernel(x, W_in, b_in, W_out, b_out, codebook):
    x2d = x.reshape(T, D)
    bin2d = b_in.reshape(1, E)
    bout2d = b_out.reshape(1, D)

    cbb, cn_col, cbw = pl.pallas_call(
        _prep_body,
        grid=(K // KT,),
        in_specs=[
            pl.BlockSpec((KT, E), lambda i: (i, 0)),
            pl.BlockSpec((D, E), lambda i: (0, 0)),
            pl.BlockSpec((1, D), lambda i: (0, 0)),
        ],
        out_specs=[
            pl.BlockSpec((KT, E), lambda i: (i, 0)),
            pl.BlockSpec((KT, 1), lambda i: (i, 0)),
            pl.BlockSpec((KT, D), lambda i: (i, 0)),
        ],
        out_shape=[
            jax.ShapeDtypeStruct((K, E), jnp.bfloat16),
            jax.ShapeDtypeStruct((K, 1), jnp.float32),
            jax.ShapeDtypeStruct((K, D), jnp.float32),
        ],
    )(codebook, W_out, bout2d)

    cn_row = cn_col.reshape(1, K)

    idx2d, aux = pl.pallas_call(
        _vq_body,
        grid=(T // TM,),
        in_specs=[
            pl.BlockSpec((TM, D), lambda i: (i, 0)),
            pl.BlockSpec((E, D), lambda i: (0, 0)),
            pl.BlockSpec((1, E), lambda i: (0, 0)),
            pl.BlockSpec((K, E), lambda i: (0, 0)),
            pl.BlockSpec((1, K), lambda i: (0, 0)),
        ],
        out_specs=[
            pl.BlockSpec((TM, 1), lambda i: (i, 0)),
            pl.BlockSpec((1, 1), lambda i: (0, 0)),
        ],
        out_shape=[
            jax.ShapeDtypeStruct((T, 1), jnp.int32),
            jax.ShapeDtypeStruct((1, 1), jnp.float32),
        ],
        scratch_shapes=[pltpu.SMEM((1,), jnp.float32)],
        compiler_params=pltpu.CompilerParams(
            dimension_semantics=("arbitrary",),
        ),
    )(x2d, W_in, bin2d, cbb, cn_row)

    idx1d = idx2d.reshape(T)

    gather2 = pl.kernel(
        _gather2_body,
        out_type=[
            jax.ShapeDtypeStruct((T, E), jnp.float32),
            jax.ShapeDtypeStruct((T, D), jnp.float32),
        ],
        mesh=plsc.VectorSubcoreMesh(
            core_axis_name="c", subcore_axis_name="s",
            num_cores=NC, num_subcores=NS,
        ),
        scratch_types=(
            [pltpu.VMEM((TW,), jnp.int32)]
            + [pltpu.VMEM((CH, E), jnp.float32) for _ in range(NBUF)]
            + [pltpu.VMEM((CH, D), jnp.float32) for _ in range(NBUF)]
            + [pltpu.SemaphoreType.DMA for _ in range(2 * NBUF)]
        ),
    )
    quant2d, out2d = gather2(idx1d, codebook, cbw)

    out = out2d.reshape(B, C, N, D)
    indices = idx1d.reshape(B, C * N)
    quantized = quant2d.reshape(B, C, N, E)
    aux_loss = aux.reshape(())
    return out, indices, quantized, aux_loss
